# baseline (speedup 1.0000x reference)
import functools

import jax
import jax.numpy as jnp
from jax import lax
from jax.experimental import pallas as pl
from jax.experimental.pallas import tpu as pltpu
from jax.experimental.pallas import tpu_sc as plsc

NC = 2
NS = 16
L = 16
NW = NC * NS

D = 64
B = 16384
V = 1000000
BPW = B // NW

TC_TOTAL = V // 128
TCPW = TC_TOTAL // NW
CHUNK_TC = 4
CHUNK_COLS = CHUNK_TC * 128
NCHUNK = TCPW // CHUNK_TC
GRP = B // L

SLOTS = 32

_mesh = plsc.VectorSubcoreMesh(core_axis_name="c", subcore_axis_name="s")


def _scan_and_sweep(wid, idx_v, tab_hbm, outflat_hbm, wl_v, chunk_v, stage_v,
                    st_m, st_p, zdrain_v, sem, semo):
    c_lo = wid * (TCPW * 128)
    c_hi = c_lo + TCPW * 128
    tail0 = TCPW * NW * 128
    e_lo = jnp.where(wid < 4, tail0 + wid * 128,
                     jnp.where(wid == 4, jnp.int32(TC_TOTAL * 128),
                               jnp.int32(1 << 30)))
    e_hi = jnp.where(wid < 4, e_lo + 128,
                     jnp.where(wid == 4, jnp.int32(V), jnp.int32(1 << 30)))

    def scan_body(j, cnt):
        grp = idx_v[pl.ds(j * L, L)]
        m = ((grp >= c_lo) & (grp < c_hi)) | ((grp >= e_lo) & (grp < e_hi))
        pos = j * L + lax.iota(jnp.int32, L)
        plsc.store_compressed(wl_v.at[pl.ds(cnt, L)], pos, mask=m)
        return cnt + jnp.sum(m.astype(jnp.int32))

    cnt = lax.fori_loop(0, GRP, scan_body, jnp.int32(0))
    wl_v[pl.ds(cnt, L)] = jnp.zeros((L,), jnp.int32)
    ngrp = (cnt + L - 1) // L

    def process_chunk(c0, chunk_valid_cols, carry):
        used, n_out = carry

        def grp_body(j, car):
            used, n_out = car
            pos16 = wl_v[pl.ds(j * L, L)]
            val16 = plsc.load_gather(idx_v, [pos16])
            c16 = val16 - c0
            m = (c16 >= 0) & (c16 < chunk_valid_cols)
            pc = jnp.sum(m.astype(jnp.int32))

            def do_extract(used, n_out):
                slot = used % SLOTS
                sbase = slot * (L * D)
                tshift8 = lax.shift_right_logical(c16, 7) * 8
                cc16 = jnp.bitwise_and(c16, 127)
                lanes = lax.iota(jnp.int32, L) * D
                for d in range(D):
                    g, f = d >> 3, d & 7
                    vals = plsc.load_gather(
                        chunk_v,
                        [tshift8 + g, jnp.full((L,), f, jnp.int32), cc16],
                        mask=m)
                    plsc.store_scatter(stage_v, [sbase + lanes + d], vals,
                                       mask=m)
                mi = m.astype(jnp.int32)
                for lane in range(L):
                    @pl.when(mi[lane] == 1)
                    def _():
                        pltpu.async_copy(
                            stage_v.at[pl.ds(sbase + lane * D, D)],
                            outflat_hbm.at[pl.ds(pos16[lane] * D, D)], semo)
                used, n_out = used + 1, n_out + pc

                def do_drain(n):
                    def w(_, c):
                        pltpu.make_async_copy(
                            outflat_hbm.at[pl.ds(0, D)], zdrain_v, semo
                        ).wait()
                        return c
                    lax.fori_loop(0, n, w, 0)
                    return jnp.int32(0)

                n_out = lax.cond(used % SLOTS == SLOTS - 1, do_drain,
                                 lambda n: n, n_out)
                return used, n_out

            return lax.cond(pc > 0, do_extract, lambda u, n: (u, n),
                            used, n_out)

        return lax.fori_loop(0, ngrp, grp_body, (used, n_out))

    def chunk_body(cc, carry):
        c0 = c_lo + cc * CHUNK_COLS
        tc0 = wid * TCPW + cc * CHUNK_TC
        dmas = []
        for t in range(CHUNK_TC):
            for g in range(8):
                dmas.append(pltpu.async_copy(
                    tab_hbm.at[pl.ds(g * 8, 8), pl.ds((tc0 + t) * 128, 128)],
                    chunk_v.at[t * 8 + g], sem))
        for d_ in dmas:
            d_.wait()
        return process_chunk(c0, jnp.int32(CHUNK_COLS), carry)

    carry = lax.fori_loop(0, NCHUNK, chunk_body, (jnp.int32(0), jnp.int32(0)))

    def tail_full(carry):
        tc = TCPW * NW + wid
        dmas = [pltpu.async_copy(
            tab_hbm.at[pl.ds(g * 8, 8), pl.ds(tc * 128, 128)],
            chunk_v.at[g], sem) for g in range(8)]
        for d_ in dmas:
            d_.wait()
        return process_chunk(e_lo, jnp.int32(128), carry)

    def tail_stub(carry):
        for d in range(D):
            g, f = d >> 3, d & 7
            pltpu.async_copy(
                tab_hbm.at[d].at[pl.ds(TC_TOTAL * 128, 64)],
                chunk_v.at[g, f, pl.ds(0, 64)], sem).wait()
        return process_chunk(e_lo, jnp.int32(64), carry)

    carry = lax.cond(wid < 4, tail_full,
                     lambda c: lax.cond(wid == 4, tail_stub, lambda c2: c2, c),
                     carry)

    _, n_out = carry

    def w(_, c):
        pltpu.make_async_copy(outflat_hbm.at[pl.ds(0, D)], zdrain_v,
                              semo).wait()
        return c

    lax.fori_loop(0, n_out, w, 0)


@functools.partial(
    pl.kernel,
    out_type=(jax.ShapeDtypeStruct((B * D,), jnp.float32),
              jax.ShapeDtypeStruct((B * D,), jnp.float32)),
    mesh=_mesh,
    scratch_types=[
        pltpu.VMEM((B,), jnp.int32),
        pltpu.VMEM((B + L,), jnp.int32),
        pltpu.VMEM((CHUNK_TC * 8, 8, 128), jnp.float32),
        pltpu.VMEM((SLOTS * L * D,), jnp.float32),
        pltpu.VMEM((L,), jnp.int32),
        pltpu.VMEM((L,), jnp.int32),
        pltpu.VMEM((D,), jnp.float32),
        pltpu.SemaphoreType.DMA,
        pltpu.SemaphoreType.DMA,
    ],
    compiler_params=pltpu.CompilerParams(needs_layout_passes=False,
                                         use_tc_tiling_on_sc=True),
)
def _gather_sc(u_idx_hbm, i_idx_hbm, u_tab_hbm, i_tab_hbm, ug_hbm, ig_hbm,
               idx_v, wl_v, chunk_v, stage_v, st_m, st_p, zdrain_v, sem, semo):
    wid = lax.axis_index("s") * NC + lax.axis_index("c")
    pltpu.sync_copy(u_idx_hbm, idx_v)
    _scan_and_sweep(wid, idx_v, u_tab_hbm, ug_hbm, wl_v, chunk_v, stage_v,
                    st_m, st_p, zdrain_v, sem, semo)
    pltpu.sync_copy(i_idx_hbm, idx_v)
    _scan_and_sweep(wid, idx_v, i_tab_hbm, ig_hbm, wl_v, chunk_v, stage_v,
                    st_m, st_p, zdrain_v, sem, semo)


@functools.partial(
    pl.kernel,
    out_type=jax.ShapeDtypeStruct((B,), jnp.float32),
    mesh=_mesh,
    scratch_types=[
        pltpu.VMEM((BPW * D,), jnp.float32),
        pltpu.VMEM((BPW * D,), jnp.float32),
        pltpu.VMEM((BPW,), jnp.float32),
        pltpu.SemaphoreType.DMA,
    ],
    compiler_params=pltpu.CompilerParams(needs_layout_passes=False),
)
def _dot_sc(ug_hbm, ig_hbm, out_hbm, uv, iv, ov, sem):
    wid = lax.axis_index("s") * NC + lax.axis_index("c")
    base = wid * BPW
    cu = pltpu.async_copy(ug_hbm.at[pl.ds(base * D, BPW * D)], uv, sem)
    ci = pltpu.async_copy(ig_hbm.at[pl.ds(base * D, BPW * D)], iv, sem)
    cu.wait()
    ci.wait()

    def group_body(g, carry):
        flat = (g * L + lax.iota(jnp.int32, L)) * D
        acc = jnp.zeros((L,), jnp.float32)
        for d in range(D):
            acc = acc + (plsc.load_gather(uv, [flat + d])
                         * plsc.load_gather(iv, [flat + d]))
        ov[pl.ds(g * L, L)] = acc
        return carry

    lax.fori_loop(0, BPW // L, group_body, 0)
    pltpu.sync_copy(ov, out_hbm.at[pl.ds(base, BPW)])


def kernel(user_input, item_input, user_table, item_table):
    ug, ig = _gather_sc(user_input.astype(jnp.int32),
                        item_input.astype(jnp.int32),
                        user_table.T, item_table.T)
    out = _dot_sc(ug, ig)
    return out.reshape(B, 1)


# speedup vs baseline: 1.1416x; 1.1416x over previous
import functools

import jax
import jax.numpy as jnp
from jax import lax
from jax.experimental import pallas as pl
from jax.experimental.pallas import tpu as pltpu
from jax.experimental.pallas import tpu_sc as plsc

NC = 2
NS = 16
L = 16
NW = NC * NS

D = 64
B = 16384
V = 1000000
BPW = B // NW

TC_TOTAL = V // 128
TCPW = TC_TOTAL // NW
CHUNK_TC = 4
CHUNK_COLS = CHUNK_TC * 128
NCHUNK = TCPW // CHUNK_TC
GRP = B // L

SLOTS = 16

_mesh = plsc.VectorSubcoreMesh(core_axis_name="c", subcore_axis_name="s")


def _scan_and_sweep(wid, idx_v, tab_hbm, outflat_hbm, wl_v, chunk_v, chunk2_v,
                    stage_v, zdrain_v, sem, sem2, semo):
    c_lo = wid * (TCPW * 128)
    c_hi = c_lo + TCPW * 128
    tail0 = TCPW * NW * 128
    e_lo = jnp.where(wid < 4, tail0 + wid * 128,
                     jnp.where(wid == 4, jnp.int32(TC_TOTAL * 128),
                               jnp.int32(1 << 30)))
    e_hi = jnp.where(wid < 4, e_lo + 128,
                     jnp.where(wid == 4, jnp.int32(V), jnp.int32(1 << 30)))

    def scan_body(j, cnt):
        grp = idx_v[pl.ds(j * L, L)]
        m = ((grp >= c_lo) & (grp < c_hi)) | ((grp >= e_lo) & (grp < e_hi))
        pos = j * L + lax.iota(jnp.int32, L)
        plsc.store_compressed(wl_v.at[pl.ds(cnt, L)], pos, mask=m)
        return cnt + jnp.sum(m.astype(jnp.int32))

    cnt = lax.fori_loop(0, GRP, scan_body, jnp.int32(0))
    wl_v[pl.ds(cnt, L)] = jnp.zeros((L,), jnp.int32)
    ngrp = (cnt + L - 1) // L

    def process_chunk(c0, chunk_valid_cols, carry, buf_v):
        used, n_out = carry

        def grp_body(j, car):
            used, n_out = car
            pos16 = wl_v[pl.ds(j * L, L)]
            val16 = plsc.load_gather(idx_v, [pos16])
            c16 = val16 - c0
            m = (c16 >= 0) & (c16 < chunk_valid_cols)
            pc = jnp.sum(m.astype(jnp.int32))

            def do_extract(used, n_out):
                slot = used % SLOTS
                sbase = slot * (L * D)
                lanes = lax.iota(jnp.int32, L) * D
                for d in range(D):
                    g, f = d >> 3, d & 7
                    vals = plsc.load_gather(
                        buf_v,
                        [jnp.full((L,), g, jnp.int32),
                         jnp.full((L,), f, jnp.int32), c16],
                        mask=m)
                    plsc.store_scatter(stage_v, [sbase + lanes + d], vals,
                                       mask=m)
                mi = m.astype(jnp.int32)
                for lane in range(L):
                    @pl.when(mi[lane] == 1)
                    def _():
                        pltpu.async_copy(
                            stage_v.at[pl.ds(sbase + lane * D, D)],
                            outflat_hbm.at[pl.ds(pos16[lane] * D, D)], semo)
                used, n_out = used + 1, n_out + pc

                def do_drain(n):
                    def w(_, c):
                        pltpu.make_async_copy(
                            outflat_hbm.at[pl.ds(0, D)], zdrain_v, semo
                        ).wait()
                        return c
                    lax.fori_loop(0, n, w, 0)
                    return jnp.int32(0)

                n_out = lax.cond(used % SLOTS == SLOTS - 1, do_drain,
                                 lambda n: n, n_out)
                return used, n_out

            return lax.cond(pc > 0, do_extract, lambda u, n: (u, n),
                            used, n_out)

        return lax.fori_loop(0, ngrp, grp_body, (used, n_out))

    def fire(cc, buf_v, s):
        col0 = (wid * TCPW + cc * CHUNK_TC) * 128
        for g in range(8):
            pltpu.async_copy(
                tab_hbm.at[pl.ds(g * 8, 8), pl.ds(col0, CHUNK_COLS)],
                buf_v.at[g], s)

    def drain8(buf_v, s):
        for g in range(8):
            pltpu.make_async_copy(
                tab_hbm.at[pl.ds(0, 8), pl.ds(0, CHUNK_COLS)],
                buf_v.at[g], s).wait()

    fire(jnp.int32(0), chunk_v, sem)
    carry = (jnp.int32(0), jnp.int32(0))

    def body(i, carry):
        fire(2 * i + 1, chunk2_v, sem2)
        drain8(chunk_v, sem)
        carry = process_chunk(c_lo + (2 * i) * CHUNK_COLS,
                              jnp.int32(CHUNK_COLS), carry, chunk_v)
        fire(2 * i + 2, chunk_v, sem)
        drain8(chunk2_v, sem2)
        carry = process_chunk(c_lo + (2 * i + 1) * CHUNK_COLS,
                              jnp.int32(CHUNK_COLS), carry, chunk2_v)
        return carry

    carry = lax.fori_loop(0, (NCHUNK - 1) // 2, body, carry)
    drain8(chunk_v, sem)
    carry = process_chunk(c_lo + (NCHUNK - 1) * CHUNK_COLS,
                          jnp.int32(CHUNK_COLS), carry, chunk_v)

    def tail_full(carry):
        tc = TCPW * NW + wid
        dmas = [pltpu.async_copy(
            tab_hbm.at[pl.ds(g * 8, 8), pl.ds(tc * 128, 128)],
            chunk_v.at[g, :, pl.ds(0, 128)], sem) for g in range(8)]
        for d_ in dmas:
            d_.wait()
        return process_chunk(e_lo, jnp.int32(128), carry, chunk_v)

    def tail_stub(carry):
        for d in range(D):
            g, f = d >> 3, d & 7
            pltpu.async_copy(
                tab_hbm.at[d].at[pl.ds(TC_TOTAL * 128, 64)],
                chunk_v.at[g, f, pl.ds(0, 64)], sem).wait()
        return process_chunk(e_lo, jnp.int32(64), carry, chunk_v)

    carry = lax.cond(wid < 4, tail_full,
                     lambda c: lax.cond(wid == 4, tail_stub, lambda c2: c2, c),
                     carry)

    _, n_out = carry

    def w(_, c):
        pltpu.make_async_copy(outflat_hbm.at[pl.ds(0, D)], zdrain_v,
                              semo).wait()
        return c

    lax.fori_loop(0, n_out, w, 0)


@functools.partial(
    pl.kernel,
    out_type=(jax.ShapeDtypeStruct((B * D,), jnp.float32),
              jax.ShapeDtypeStruct((B * D,), jnp.float32)),
    mesh=_mesh,
    scratch_types=[
        pltpu.VMEM((B,), jnp.int32),
        pltpu.VMEM((B + L,), jnp.int32),
        pltpu.VMEM((8, 8, CHUNK_COLS), jnp.float32),
        pltpu.VMEM((8, 8, CHUNK_COLS), jnp.float32),
        pltpu.VMEM((SLOTS * L * D,), jnp.float32),
        pltpu.VMEM((D,), jnp.float32),
        pltpu.SemaphoreType.DMA,
        pltpu.SemaphoreType.DMA,
        pltpu.SemaphoreType.DMA,
    ],
    compiler_params=pltpu.CompilerParams(needs_layout_passes=False,
                                         use_tc_tiling_on_sc=True),
)
def _gather_sc(u_idx_hbm, i_idx_hbm, u_tab_hbm, i_tab_hbm, ug_hbm, ig_hbm,
               idx_v, wl_v, chunk_v, chunk2_v, stage_v, zdrain_v,
               sem, sem2, semo):
    wid = lax.axis_index("s") * NC + lax.axis_index("c")
    pltpu.sync_copy(u_idx_hbm, idx_v)
    _scan_and_sweep(wid, idx_v, u_tab_hbm, ug_hbm, wl_v, chunk_v, chunk2_v,
                    stage_v, zdrain_v, sem, sem2, semo)
    pltpu.sync_copy(i_idx_hbm, idx_v)
    _scan_and_sweep(wid, idx_v, i_tab_hbm, ig_hbm, wl_v, chunk_v, chunk2_v,
                    stage_v, zdrain_v, sem, sem2, semo)


@functools.partial(
    pl.kernel,
    out_type=jax.ShapeDtypeStruct((B,), jnp.float32),
    mesh=_mesh,
    scratch_types=[
        pltpu.VMEM((BPW * D,), jnp.float32),
        pltpu.VMEM((BPW * D,), jnp.float32),
        pltpu.VMEM((BPW,), jnp.float32),
        pltpu.SemaphoreType.DMA,
    ],
    compiler_params=pltpu.CompilerParams(needs_layout_passes=False),
)
def _dot_sc(ug_hbm, ig_hbm, out_hbm, uv, iv, ov, sem):
    wid = lax.axis_index("s") * NC + lax.axis_index("c")
    base = wid * BPW
    cu = pltpu.async_copy(ug_hbm.at[pl.ds(base * D, BPW * D)], uv, sem)
    ci = pltpu.async_copy(ig_hbm.at[pl.ds(base * D, BPW * D)], iv, sem)
    cu.wait()
    ci.wait()

    def group_body(g, carry):
        flat = (g * L + lax.iota(jnp.int32, L)) * D
        acc = jnp.zeros((L,), jnp.float32)
        for d in range(D):
            acc = acc + (plsc.load_gather(uv, [flat + d])
                         * plsc.load_gather(iv, [flat + d]))
        ov[pl.ds(g * L, L)] = acc
        return carry

    lax.fori_loop(0, BPW // L, group_body, 0)
    pltpu.sync_copy(ov, out_hbm.at[pl.ds(base, BPW)])


def kernel(user_input, item_input, user_table, item_table):
    ug, ig = _gather_sc(user_input.astype(jnp.int32),
                        item_input.astype(jnp.int32),
                        user_table.T, item_table.T)
    out = _dot_sc(ug, ig)
    return out.reshape(B, 1)


# speedup vs baseline: 4.4523x; 3.9001x over previous
import functools

import jax
import jax.numpy as jnp
from jax import lax
from jax.experimental import pallas as pl
from jax.experimental.pallas import tpu as pltpu
from jax.experimental.pallas import tpu_sc as plsc

NC = 2
NS = 16
L = 16
NW = NC * NS

D = 64
B = 16384
V = 1000000
BPW = B // NW

TC_TOTAL = V // 128
TCPW = TC_TOTAL // NW
CHUNK_TC = 4
CHUNK_COLS = CHUNK_TC * 128
NCHUNK = TCPW // CHUNK_TC
GRP = B // L

SLOTS = 8

_mesh = plsc.VectorSubcoreMesh(core_axis_name="c", subcore_axis_name="s",
                               num_cores=NC, num_subcores=NS)


def _scan_and_sweep(wid, idx_v, tab_hbm, outflat_hbm, wl_v, wl2_v, chunk_v,
                    chunk2_v, stage_v, zdrain_v, sem, sem2, semo):
    c_lo = wid * (TCPW * 128)
    c_hi = c_lo + TCPW * 128
    tail0 = TCPW * NW * 128
    e_lo = jnp.where(wid < 4, tail0 + wid * 128,
                     jnp.where(wid == 4, jnp.int32(TC_TOTAL * 128),
                               jnp.int32(1 << 30)))
    e_hi = jnp.where(wid < 4, e_lo + 128,
                     jnp.where(wid == 4, jnp.int32(V), jnp.int32(1 << 30)))

    def scan_body(j, cnt):
        grp = idx_v[pl.ds(j * L, L)]
        m = ((grp >= c_lo) & (grp < c_hi)) | ((grp >= e_lo) & (grp < e_hi))
        pos = j * L + lax.iota(jnp.int32, L)
        plsc.store_compressed(wl_v.at[pl.ds(cnt, L)], pos, mask=m)
        return cnt + jnp.sum(m.astype(jnp.int32))

    cnt = lax.fori_loop(0, GRP, scan_body, jnp.int32(0))
    ngrp = (cnt + L - 1) // L

    def bucket_of(val16):
        b = lax.shift_right_logical(val16 - c_lo, 12)
        return jnp.minimum(b, 7)

    def count_body(j, counts):
        vmask = (j * L + lax.iota(jnp.int32, L)) < cnt
        pos16 = jnp.where(vmask, wl_v[pl.ds(j * L, L)], 0)
        val16 = plsc.load_gather(idx_v, [pos16])
        b16 = bucket_of(val16)
        return tuple(
            counts[k] + jnp.sum(((b16 == k) & vmask).astype(jnp.int32))
            for k in range(8))

    counts = lax.fori_loop(0, ngrp, count_body, (jnp.int32(0),) * 8)
    starts = []
    acc = jnp.int32(0)
    for k in range(8):
        starts.append(acc)
        acc = acc + counts[k]

    def place_body(j, offs):
        vmask = (j * L + lax.iota(jnp.int32, L)) < cnt
        pos16 = jnp.where(vmask, wl_v[pl.ds(j * L, L)], 0)
        val16 = plsc.load_gather(idx_v, [pos16])
        b16 = bucket_of(val16)
        new = []
        for k in range(8):
            mk = (b16 == k) & vmask
            plsc.store_compressed(wl2_v.at[pl.ds(offs[k], L)], pos16, mask=mk)
            new.append(offs[k] + jnp.sum(mk.astype(jnp.int32)))
        return tuple(new)

    lax.fori_loop(0, ngrp, place_body, tuple(starts))
    wl2_v[pl.ds(cnt, L)] = jnp.zeros((L,), jnp.int32)

    def bucket_span(k):
        bs, bc = starts[7], counts[7]
        for kk in range(6, -1, -1):
            bs = jnp.where(k == kk, starts[kk], bs)
            bc = jnp.where(k == kk, counts[kk], bc)
        return bs, bc

    def process_chunk(c0, chunk_valid_cols, k, carry, buf_v):
        bs, bc = bucket_span(k)
        bgrp = (bc + L - 1) // L

        def grp_body(j, car):
            used, n_out = car
            pos16 = wl2_v[pl.ds(bs + j * L, L)]
            val16 = plsc.load_gather(idx_v, [pos16])
            c16 = val16 - c0
            m = (c16 >= 0) & (c16 < chunk_valid_cols)
            pc = jnp.sum(m.astype(jnp.int32))

            def do_extract(used, n_out):
                slot = used % SLOTS
                sbase = slot * (L * D)
                lanes = lax.iota(jnp.int32, L) * D
                for d in range(D):
                    g, f = d >> 3, d & 7
                    vals = plsc.load_gather(
                        buf_v,
                        [jnp.full((L,), g, jnp.int32),
                         jnp.full((L,), f, jnp.int32), c16],
                        mask=m)
                    plsc.store_scatter(stage_v, [sbase + lanes + d], vals,
                                       mask=m)
                mi = m.astype(jnp.int32)
                for lane in range(L):
                    @pl.when(mi[lane] == 1)
                    def _():
                        pltpu.async_copy(
                            stage_v.at[pl.ds(sbase + lane * D, D)],
                            outflat_hbm.at[pl.ds(pos16[lane] * D, D)], semo)
                used, n_out = used + 1, n_out + pc

                def do_drain(n):
                    def w(_, c):
                        pltpu.make_async_copy(
                            outflat_hbm.at[pl.ds(0, D)], zdrain_v, semo
                        ).wait()
                        return c
                    lax.fori_loop(0, n, w, 0)
                    return jnp.int32(0)

                n_out = lax.cond(used % SLOTS == 0, do_drain,
                                 lambda n: n, n_out)
                return used, n_out

            return lax.cond(pc > 0, do_extract, lambda u, n: (u, n),
                            used, n_out)

        return lax.fori_loop(0, bgrp, grp_body, carry)

    def fire(cc, buf_v, s):
        col0 = (wid * TCPW + cc * CHUNK_TC) * 128
        for g in range(8):
            pltpu.async_copy(
                tab_hbm.at[pl.ds(g * 8, 8), pl.ds(col0, CHUNK_COLS)],
                buf_v.at[g], s)

    def drain8(buf_v, s):
        for g in range(8):
            pltpu.make_async_copy(
                tab_hbm.at[pl.ds(0, 8), pl.ds(0, CHUNK_COLS)],
                buf_v.at[g], s).wait()

    fire(jnp.int32(0), chunk_v, sem)
    carry = (jnp.int32(0), jnp.int32(0))

    def body(i, carry):
        cc = 2 * i
        fire(cc + 1, chunk2_v, sem2)
        drain8(chunk_v, sem)
        carry = process_chunk(c_lo + cc * CHUNK_COLS, jnp.int32(CHUNK_COLS),
                              lax.shift_right_logical(cc, 3), carry, chunk_v)
        fire(cc + 2, chunk_v, sem)
        drain8(chunk2_v, sem2)
        carry = process_chunk(c_lo + (cc + 1) * CHUNK_COLS,
                              jnp.int32(CHUNK_COLS),
                              lax.shift_right_logical(cc + 1, 3), carry,
                              chunk2_v)
        return carry

    carry = lax.fori_loop(0, (NCHUNK - 1) // 2, body, carry)
    drain8(chunk_v, sem)
    carry = process_chunk(c_lo + (NCHUNK - 1) * CHUNK_COLS,
                          jnp.int32(CHUNK_COLS), jnp.int32(7), carry, chunk_v)

    def tail_full(carry):
        tc = TCPW * NW + wid
        dmas = [pltpu.async_copy(
            tab_hbm.at[pl.ds(g * 8, 8), pl.ds(tc * 128, 128)],
            chunk_v.at[g, :, pl.ds(0, 128)], sem) for g in range(8)]
        for d_ in dmas:
            d_.wait()
        return process_chunk(e_lo, jnp.int32(128), jnp.int32(7), carry,
                             chunk_v)

    def tail_stub(carry):
        for d in range(D):
            g, f = d >> 3, d & 7
            pltpu.async_copy(
                tab_hbm.at[d].at[pl.ds(TC_TOTAL * 128, 64)],
                chunk_v.at[g, f, pl.ds(0, 64)], sem).wait()
        return process_chunk(e_lo, jnp.int32(64), jnp.int32(7), carry,
                             chunk_v)

    carry = lax.cond(wid < 4, tail_full,
                     lambda c: lax.cond(wid == 4, tail_stub, lambda c2: c2, c),
                     carry)

    def w(_, c):
        pltpu.make_async_copy(outflat_hbm.at[pl.ds(0, D)], zdrain_v,
                              semo).wait()
        return c

    lax.fori_loop(0, carry[1], w, 0)


@functools.partial(
    pl.kernel,
    out_type=(jax.ShapeDtypeStruct((B * D + D,), jnp.float32),
              jax.ShapeDtypeStruct((B * D + D,), jnp.float32)),
    mesh=_mesh,
    scratch_types=[
        pltpu.VMEM((B,), jnp.int32),
        pltpu.VMEM((B + L,), jnp.int32),
        pltpu.VMEM((B + L,), jnp.int32),
        pltpu.VMEM((8, 8, CHUNK_COLS), jnp.float32),
        pltpu.VMEM((8, 8, CHUNK_COLS), jnp.float32),
        pltpu.VMEM((SLOTS * L * D,), jnp.float32),
        pltpu.VMEM((D,), jnp.float32),
        pltpu.SemaphoreType.DMA,
        pltpu.SemaphoreType.DMA,
        pltpu.SemaphoreType.DMA,
    ],
    compiler_params=pltpu.CompilerParams(needs_layout_passes=False,
                                         use_tc_tiling_on_sc=True),
)
def _gather_sc(u_idx_hbm, i_idx_hbm, u_tab_hbm, i_tab_hbm, ug_hbm, ig_hbm,
               idx_v, wl_v, wl2_v, chunk_v, chunk2_v, stage_v, zdrain_v,
               sem, sem2, semo):
    wid = lax.axis_index("s") * NC + lax.axis_index("c")
    pltpu.sync_copy(u_idx_hbm, idx_v)
    _scan_and_sweep(wid, idx_v, u_tab_hbm, ug_hbm, wl_v, wl2_v, chunk_v,
                    chunk2_v, stage_v, zdrain_v, sem, sem2, semo)
    pltpu.sync_copy(i_idx_hbm, idx_v)
    _scan_and_sweep(wid, idx_v, i_tab_hbm, ig_hbm, wl_v, wl2_v, chunk_v,
                    chunk2_v, stage_v, zdrain_v, sem, sem2, semo)


@functools.partial(
    pl.kernel,
    out_type=jax.ShapeDtypeStruct((B,), jnp.float32),
    mesh=_mesh,
    scratch_types=[
        pltpu.VMEM((BPW * D,), jnp.float32),
        pltpu.VMEM((BPW * D,), jnp.float32),
        pltpu.VMEM((BPW,), jnp.float32),
        pltpu.SemaphoreType.DMA,
    ],
    compiler_params=pltpu.CompilerParams(needs_layout_passes=False),
)
def _dot_sc(ug_hbm, ig_hbm, out_hbm, uv, iv, ov, sem):
    wid = lax.axis_index("s") * NC + lax.axis_index("c")
    base = wid * BPW
    cu = pltpu.async_copy(ug_hbm.at[pl.ds(base * D, BPW * D)], uv, sem)
    ci = pltpu.async_copy(ig_hbm.at[pl.ds(base * D, BPW * D)], iv, sem)
    cu.wait()
    ci.wait()

    def group_body(g, carry):
        flat = (g * L + lax.iota(jnp.int32, L)) * D
        acc = jnp.zeros((L,), jnp.float32)
        for d in range(D):
            acc = acc + (plsc.load_gather(uv, [flat + d])
                         * plsc.load_gather(iv, [flat + d]))
        ov[pl.ds(g * L, L)] = acc
        return carry

    lax.fori_loop(0, BPW // L, group_body, 0)
    pltpu.sync_copy(ov, out_hbm.at[pl.ds(base, BPW)])


def kernel(user_input, item_input, user_table, item_table):
    ug, ig = _gather_sc(user_input.astype(jnp.int32),
                        item_input.astype(jnp.int32),
                        user_table.T, item_table.T)
    out = _dot_sc(ug, ig)
    return out.reshape(B, 1)
